# Initial kernel scaffold; baseline (speedup 1.0000x reference)
import jax
import jax.numpy as jnp
from jax.experimental import pallas as pl


def kernel(t, x, edge_index):
    raise NotImplementedError("write your pallas kernel here")


# speedup vs baseline: 3.6084x; 3.6084x over previous
import jax
import jax.numpy as jnp
from jax import lax
from jax.experimental import pallas as pl
from jax.experimental.pallas import tpu as pltpu
from jax.experimental.pallas import tpu_sc as plsc

N_NODES = 10000
N_EDGES = 320000
D = 128

NC = 2
NS = 16
NW = NC * NS
CHUNK = 128
CHUNKS_PER_W = 80
E_PAD = NW * CHUNKS_PER_W * CHUNK
ACC_ROWS = 10240
PAD_ROW = N_NODES + 8
ZCOPIES = 5
IDX_DEPTH = 4
ROWS_PER_TILE_OUT = 624
OUT_TAIL = N_NODES - NS * ROWS_PER_TILE_OUT


def _sc_body(x_hbm, eidx_hbm, out_hbm,
             acc_sh, idxs, rows,
             gsem0, gsem1, isem0, isem1, isem2, isem3):
    gsem = [gsem0, gsem1]
    isem = [isem0, isem1, isem2, isem3]
    c = lax.axis_index("c")
    s = lax.axis_index("s")
    base = (c * NS + s) * CHUNKS_PER_W

    def _zrow(r, _):
        for k in range(D // 16):
            rows[0, r, pl.ds(16 * k, 16)] = jnp.zeros((16,), jnp.float32)
        return 0
    lax.fori_loop(0, CHUNK, _zrow, 0)
    for q in range(ZCOPIES):
        pltpu.sync_copy(rows.at[0],
                        acc_sh.at[pl.ds(s * (ZCOPIES * CHUNK) + q * CHUNK, CHUNK)])

    pltpu.sync_copy(eidx_hbm.at[base], idxs.at[0])
    pltpu.async_copy(x_hbm.at[idxs.at[0, 0]], rows.at[0], gsem[0])
    for p in range(1, IDX_DEPTH):
        pltpu.async_copy(eidx_hbm.at[base + p], idxs.at[p], isem[p])

    plsc.subcore_barrier()

    def _quad(i, _):
        for b in range(IDX_DEPTH):
            jb = IDX_DEPTH * i + b
            pn = (b + 1) % IDX_DEPTH
            rn = (b + 1) % 2
            @pl.when(jb + 1 < CHUNKS_PER_W)
            def _():
                pltpu.make_async_copy(eidx_hbm.at[base], idxs.at[pn],
                                      isem[pn]).wait()
                pltpu.async_copy(x_hbm.at[idxs.at[pn, 0]], rows.at[rn],
                                 gsem[rn])
            pltpu.make_async_copy(x_hbm.at[pl.ds(0, CHUNK)], rows.at[b % 2],
                                  gsem[b % 2]).wait()
            pltpu.sync_copy(rows.at[b % 2], acc_sh.at[idxs.at[b, 1]], add=True)
            @pl.when(jb + IDX_DEPTH < CHUNKS_PER_W)
            def _():
                pltpu.async_copy(eidx_hbm.at[base + jb + IDX_DEPTH],
                                 idxs.at[b], isem[b])
        return 0

    lax.fori_loop(0, CHUNKS_PER_W // IDX_DEPTH, _quad, 0)

    plsc.subcore_barrier()

    pltpu.sync_copy(acc_sh.at[pl.ds(s * ROWS_PER_TILE_OUT, ROWS_PER_TILE_OUT)],
                    out_hbm.at[c, pl.ds(s * ROWS_PER_TILE_OUT, ROWS_PER_TILE_OUT)])

    @pl.when(s == 0)
    def _():
        pltpu.sync_copy(acc_sh.at[pl.ds(NS * ROWS_PER_TILE_OUT, OUT_TAIL)],
                        out_hbm.at[c, pl.ds(NS * ROWS_PER_TILE_OUT, OUT_TAIL)])


def _tc_add_body(p_ref, o_ref):
    o_ref[...] = p_ref[0] + p_ref[1]


@jax.jit
def _run(x, edge_index):
    n_pad = E_PAD - N_EDGES
    src_p = jnp.concatenate([edge_index[0], jnp.zeros((n_pad,), jnp.int32)])
    dst_p = jnp.concatenate([edge_index[1], jnp.full((n_pad,), PAD_ROW, jnp.int32)])
    eidx = jnp.stack([src_p.reshape(NW * CHUNKS_PER_W, CHUNK),
                      dst_p.reshape(NW * CHUNKS_PER_W, CHUNK)], axis=1)

    mesh = plsc.VectorSubcoreMesh(core_axis_name="c", subcore_axis_name="s")
    partials = pl.kernel(
        _sc_body,
        out_type=jax.ShapeDtypeStruct((NC, N_NODES, D), jnp.float32),
        mesh=mesh,
        scratch_types=[
            pltpu.VMEM_SHARED((ACC_ROWS, D), jnp.float32),
            pltpu.VMEM((IDX_DEPTH, 2, CHUNK), jnp.int32),
            pltpu.VMEM((2, CHUNK, D), jnp.float32),
            pltpu.SemaphoreType.DMA,
            pltpu.SemaphoreType.DMA,
            pltpu.SemaphoreType.DMA,
            pltpu.SemaphoreType.DMA,
            pltpu.SemaphoreType.DMA,
            pltpu.SemaphoreType.DMA,
        ],
    )(x, eidx)

    out = pl.pallas_call(
        _tc_add_body,
        out_shape=jax.ShapeDtypeStruct((N_NODES, D), jnp.float32),
        grid=(10,),
        in_specs=[pl.BlockSpec((NC, N_NODES // 10, D), lambda i: (0, i, 0))],
        out_specs=pl.BlockSpec((N_NODES // 10, D), lambda i: (i, 0)),
    )(partials)
    return out


def kernel(t, x, edge_index):
    return _run(x, edge_index)
